# baseline (speedup 1.0000x reference)
import functools

import jax
import jax.numpy as jnp
from jax import lax
from jax.experimental import pallas as pl
from jax.experimental.pallas import tpu as pltpu
from jax.experimental.pallas import tpu_sc as plsc

NN = 10000
EE = 320000
NC = 2
NS = 16
HW = 64
CK = 128
NB = 5
NPAD = 10240
RPT = NPAD // NS
TAIL = NN - (NS - 1) * RPT
ZR = 64
DEGW = 16
NCHW = 160
NCHE = 80
EPAD = NS * NCHW * CK
DUMP = NN + 16


def _zero_vmem(buf, rows, w):
    def _zrow(r, carry):
        for cc in range(w // 16):
            buf[r, pl.ds(cc * 16, 16)] = jnp.zeros((16,), jnp.float32)
        return carry
    lax.fori_loop(0, rows, _zrow, 0)


def _copy_out(acc, out_ref, s):
    @pl.when(s < NS - 1)
    def _full():
        pltpu.sync_copy(acc.at[pl.ds(s * RPT, RPT)],
                        out_ref.at[pl.ds(s * RPT, RPT)])

    @pl.when(s == NS - 1)
    def _tail():
        pltpu.sync_copy(acc.at[pl.ds((NS - 1) * RPT, TAIL)],
                        out_ref.at[pl.ds((NS - 1) * RPT, TAIL)])


def _seg_body(split_cols, nch, src_hbm, dst_hbm, h_hbm, out_hbm,
              src_all, dst_all, r0, r1, r2, r3, r4, zbuf, acc, sem_g, sem_s):
    rows = (r0, r1, r2, r3, r4)
    cid = lax.axis_index("c")
    s = lax.axis_index("s")
    tid = s if split_cols else cid * NS + s
    hsrc = h_hbm.at[cid] if split_cols else h_hbm
    w = HW if split_cols else 16

    _zero_vmem(zbuf, ZR, w)
    zcopies = []
    for i in range(RPT // ZR):
        zcopies.append(pltpu.async_copy(
            zbuf, acc.at[pl.ds(s * RPT + i * ZR, ZR)], sem_g.at[i % NB]))
    zcopies.append(pltpu.async_copy(src_hbm.at[tid], src_all, sem_s.at[0]))
    zcopies.append(pltpu.async_copy(dst_hbm.at[tid], dst_all, sem_s.at[1]))
    for cp in zcopies:
        cp.wait()
    plsc.subcore_barrier()

    def _issue_gather(c, b):
        pltpu.async_copy(hsrc.at[src_all.at[c]], rows[b], sem_g.at[b])

    def _wait_gather(b):
        pltpu.make_async_copy(hsrc.at[src_all.at[0]], rows[b],
                              sem_g.at[b]).wait()

    def _issue_scatter(c, b):
        pltpu.async_copy(rows[b], acc.at[dst_all.at[c]], sem_s.at[b],
                         add=True)

    def _wait_scatter(b):
        pltpu.make_async_copy(rows[b], acc.at[dst_all.at[0]],
                              sem_s.at[b]).wait()

    def _group(j, carry):
        for b in range(NB):
            c = j * NB + b

            @pl.when(j >= 1)
            def _ws():
                _wait_scatter(b)
            _issue_gather(c, b)

            pb = (b - 1) % NB
            if b == 0:
                @pl.when(j >= 1)
                def _sc():
                    _wait_gather(pb)
                    _issue_scatter(j * NB - 1, pb)
            else:
                _wait_gather(pb)
                _issue_scatter(c - 1, pb)
        return carry
    lax.fori_loop(0, nch // NB, _group, 0)

    _wait_gather(NB - 1)
    _issue_scatter(nch - 1, NB - 1)
    for b in range(NB):
        _wait_scatter(b)
    plsc.subcore_barrier()

    _copy_out(acc, out_hbm.at[cid], s)


def _deg_body(dst_hbm, deg_hbm, dst_all, ones_v, zbuf, deg_acc, sem):
    cid = lax.axis_index("c")
    s = lax.axis_index("s")
    tid = cid * NS + s

    _zero_vmem(zbuf, ZR, DEGW)

    def _orow(r, carry):
        ones_v[r, pl.ds(0, 16)] = jnp.ones((16,), jnp.float32)
        return carry
    lax.fori_loop(0, CK, _orow, 0)

    zcopies = [pltpu.async_copy(dst_hbm.at[tid], dst_all, sem.at[0])]
    for i in range(RPT // ZR):
        zcopies.append(pltpu.async_copy(
            zbuf, deg_acc.at[pl.ds(s * RPT + i * ZR, ZR)],
            sem.at[1 + i % (NB - 1)]))
    for cp in zcopies:
        cp.wait()
    plsc.subcore_barrier()

    def _chunk(j, carry):
        @pl.when(j >= NB)
        def _w():
            pltpu.make_async_copy(ones_v, deg_acc.at[dst_all.at[0]],
                                  sem.at[0]).wait()
        pltpu.async_copy(ones_v, deg_acc.at[dst_all.at[j]], sem.at[0],
                         add=True)
        return carry
    lax.fori_loop(0, NCHE, _chunk, 0)
    for _ in range(NB):
        pltpu.make_async_copy(ones_v, deg_acc.at[dst_all.at[0]],
                              sem.at[0]).wait()
    plsc.subcore_barrier()
    _copy_out(deg_acc, deg_hbm.at[cid], s)


_SC_MESH = plsc.VectorSubcoreMesh(core_axis_name="c", subcore_axis_name="s",
                                  num_cores=NC, num_subcores=NS)
_SC_PARAMS = pltpu.CompilerParams(use_tc_tiling_on_sc=False)


def _make_seg(split_cols, nch, w):
    return pl.kernel(
        functools.partial(_seg_body, split_cols, nch),
        out_type=[jax.ShapeDtypeStruct((NC, NN, w), jnp.float32)],
        mesh=_SC_MESH,
        scratch_types=[
            pltpu.VMEM((nch, CK), jnp.int32),
            pltpu.VMEM((nch, CK), jnp.int32),
        ] + [pltpu.VMEM((CK, w), jnp.float32) for _ in range(NB)] + [
            pltpu.VMEM((ZR, w), jnp.float32),
            pltpu.VMEM_SHARED((NPAD, w), jnp.float32),
            pltpu.SemaphoreType.DMA((NB,)),
            pltpu.SemaphoreType.DMA((NB,)),
        ],
        compiler_params=_SC_PARAMS)


_segw = _make_seg(True, NCHW, HW)
_sege = _make_seg(False, NCHE, 16)

_degk = pl.kernel(
    _deg_body,
    out_type=[jax.ShapeDtypeStruct((NC, NN, DEGW), jnp.float32)],
    mesh=_SC_MESH,
    scratch_types=[
        pltpu.VMEM((NCHE, CK), jnp.int32),
        pltpu.VMEM((CK, DEGW), jnp.float32),
        pltpu.VMEM((ZR, DEGW), jnp.float32),
        pltpu.VMEM_SHARED((NPAD, DEGW), jnp.float32),
        pltpu.SemaphoreType.DMA((NB,)),
    ],
    compiler_params=_SC_PARAMS)

BM = 1000


def _dot(a, b):
    return jnp.dot(a, b, preferred_element_type=jnp.float32)


def _mm_body(wn3_ref_last, h_ref, a_ref, d_ref, ws_ref, wn_ref, b_ref,
             out_ref, p3_ref=None):
    deg = jnp.maximum(d_ref[0, :, 0:1] + d_ref[1, :, 0:1], 1.0)
    hself = _dot(h_ref[0], ws_ref[:HW, :]) + _dot(h_ref[1], ws_ref[HW:, :])
    hn = (_dot(a_ref[0], wn_ref[:HW, :]) + _dot(a_ref[1], wn_ref[HW:, :])
          ) / deg
    y = jnp.maximum(hself + hn + b_ref[...], 0.0)
    out_ref[0] = y[:, :HW]
    out_ref[1] = y[:, HW:]
    if p3_ref is not None:
        p3_ref[...] = _dot(y, wn3_ref_last[...])


def _fin_body(h_ref, a_ref, d_ref, ws_ref, b_ref, out_ref):
    deg = jnp.maximum(d_ref[0, :, 0:1] + d_ref[1, :, 0:1], 1.0)
    hn = (a_ref[0] + a_ref[1]) / deg
    out_ref[...] = (_dot(h_ref[0], ws_ref[:HW, :])
                    + _dot(h_ref[1], ws_ref[HW:, :]) + hn + b_ref[...])


def _rows_spec(width):
    return pl.BlockSpec((BM, width), lambda i: (i, 0))


def _split_spec(width):
    return pl.BlockSpec((NC, BM, width), lambda i: (0, i, 0))


def _full_spec(r, width):
    return pl.BlockSpec((r, width), lambda i: (0, 0))


def _mm(h, a, d, ws, wn, b, wn3=None):
    body = functools.partial(_mm_body, None) if wn3 is None else _mm_body
    in_specs = [_split_spec(HW), _split_spec(HW), _split_spec(DEGW),
                _full_spec(128, 128), _full_spec(128, 128),
                _full_spec(1, 128)]
    out_specs = _split_spec(HW)
    out_shape = jax.ShapeDtypeStruct((NC, NN, HW), jnp.float32)
    args = [h, a, d, ws, wn, b]
    if wn3 is not None:
        in_specs = [_full_spec(128, 16)] + in_specs
        out_specs = [out_specs, _rows_spec(16)]
        out_shape = [out_shape, jax.ShapeDtypeStruct((NN, 16), jnp.float32)]
        args = [wn3] + args
    return pl.pallas_call(body, grid=(NN // BM,), in_specs=in_specs,
                          out_specs=out_specs, out_shape=out_shape)(*args)


def _fin(h, a, d, ws, b):
    return pl.pallas_call(
        _fin_body,
        grid=(NN // BM,),
        in_specs=[_split_spec(HW), _split_spec(16), _split_spec(DEGW),
                  _full_spec(128, 16), _full_spec(1, 16)],
        out_specs=_rows_spec(16),
        out_shape=jax.ShapeDtypeStruct((NN, 16), jnp.float32),
    )(h, a, d, ws, b)


def kernel(x, edge_index, W_self1, W_neigh1, b1, W_self2, W_neigh2, b2,
           W_self3, W_neigh3, b3):
    src = edge_index[0].astype(jnp.int32)
    dst = edge_index[1].astype(jnp.int32)
    src_p = jnp.concatenate([src, jnp.zeros((EPAD - EE,), jnp.int32)])
    dst_p = jnp.concatenate([dst, jnp.full((EPAD - EE,), DUMP, jnp.int32)])
    src_w = src_p.reshape(NS, NCHW, CK)
    dst_w = dst_p.reshape(NS, NCHW, CK)
    src_e = src_p.reshape(NC * NS, NCHE, CK)
    dst_e = dst_p.reshape(NC * NS, NCHE, CK)
    xs = jnp.stack([x[:, :HW], x[:, HW:]])

    (deg,) = _degk(dst_e)
    (agg1,) = _segw(src_w, dst_w, xs)
    h1 = _mm(xs, agg1, deg, W_self1, W_neigh1, b1.reshape(1, 128))
    (agg2,) = _segw(src_w, dst_w, h1)
    h2, p3 = _mm(h1, agg2, deg, W_self2, W_neigh2, b2.reshape(1, 128),
                 wn3=W_neigh3)
    (agg3,) = _sege(src_e, dst_e, p3)
    out = _fin(h2, agg3, deg, W_self3, b3.reshape(1, 16))
    return out


# speedup vs baseline: 2.0186x; 2.0186x over previous
import functools

import jax
import jax.numpy as jnp
from jax import lax
from jax.experimental import pallas as pl
from jax.experimental.pallas import tpu as pltpu
from jax.experimental.pallas import tpu_sc as plsc

NN = 10000
EE = 320000
NC = 2
NS = 16
HW = 64
CK = 128
NB = 5
NPAD = 10240
RPT = NPAD // NS
TAIL = NN - (NS - 1) * RPT
ZR = 64
DEGW = 16
NCHW = 160
NCHE = 80
EPAD = NS * NCHW * CK
DUMP = NN + 16


def _zero_vmem(buf, rows, w):
    def _zrow(r, carry):
        for cc in range(w // 16):
            buf[r, pl.ds(cc * 16, 16)] = jnp.zeros((16,), jnp.float32)
        return carry
    lax.fori_loop(0, rows, _zrow, 0)


def _copy_out(acc, out_ref, s):
    @pl.when(s < NS - 1)
    def _full():
        pltpu.sync_copy(acc.at[pl.ds(s * RPT, RPT)],
                        out_ref.at[pl.ds(s * RPT, RPT)])

    @pl.when(s == NS - 1)
    def _tail():
        pltpu.sync_copy(acc.at[pl.ds((NS - 1) * RPT, TAIL)],
                        out_ref.at[pl.ds((NS - 1) * RPT, TAIL)])


def _seg_body(split_cols, nch, src_hbm, dst_hbm, h_hbm, out_hbm,
              src_all, dst_all, r0, r1, r2, r3, r4, zbuf, acc, sem_g, sem_s):
    rows = (r0, r1, r2, r3, r4)
    cid = lax.axis_index("c")
    s = lax.axis_index("s")
    tid = s if split_cols else cid * NS + s
    hsrc = h_hbm.at[cid] if split_cols else h_hbm
    w = HW if split_cols else 16

    _zero_vmem(zbuf, ZR, w)
    zcopies = []
    for i in range(RPT // ZR):
        zcopies.append(pltpu.async_copy(
            zbuf, acc.at[pl.ds(s * RPT + i * ZR, ZR)], sem_g.at[i % NB]))
    zcopies.append(pltpu.async_copy(src_hbm.at[tid], src_all, sem_s.at[0]))
    zcopies.append(pltpu.async_copy(dst_hbm.at[tid], dst_all, sem_s.at[1]))
    for cp in zcopies:
        cp.wait()
    plsc.subcore_barrier()

    def _issue_gather(c, b):
        pltpu.async_copy(hsrc.at[src_all.at[c]], rows[b], sem_g.at[b])

    def _wait_gather(b):
        pltpu.make_async_copy(hsrc.at[src_all.at[0]], rows[b],
                              sem_g.at[b]).wait()

    def _issue_scatter(c, b):
        pltpu.async_copy(rows[b], acc.at[dst_all.at[c]], sem_s.at[b],
                         add=True)

    def _wait_scatter(b):
        pltpu.make_async_copy(rows[b], acc.at[dst_all.at[0]],
                              sem_s.at[b]).wait()

    def _group(j, carry):
        for b in range(NB):
            c = j * NB + b

            @pl.when(j >= 1)
            def _ws():
                _wait_scatter(b)
            _issue_gather(c, b)

            pb = (b - 1) % NB
            if b == 0:
                @pl.when(j >= 1)
                def _sc():
                    _wait_gather(pb)
                    _issue_scatter(j * NB - 1, pb)
            else:
                _wait_gather(pb)
                _issue_scatter(c - 1, pb)
        return carry
    lax.fori_loop(0, nch // NB, _group, 0)

    _wait_gather(NB - 1)
    _issue_scatter(nch - 1, NB - 1)
    for b in range(NB):
        _wait_scatter(b)
    plsc.subcore_barrier()

    _copy_out(acc, out_hbm.at[cid], s)


def _deg_body(dst_hbm, deg_hbm, dst_all, ones_v, zbuf, deg_acc, sem):
    cid = lax.axis_index("c")
    s = lax.axis_index("s")
    tid = cid * NS + s

    _zero_vmem(zbuf, ZR, DEGW)

    def _orow(r, carry):
        ones_v[r, pl.ds(0, 16)] = jnp.ones((16,), jnp.float32)
        return carry
    lax.fori_loop(0, CK, _orow, 0)

    zcopies = [pltpu.async_copy(dst_hbm.at[tid], dst_all, sem.at[0])]
    for i in range(RPT // ZR):
        zcopies.append(pltpu.async_copy(
            zbuf, deg_acc.at[pl.ds(s * RPT + i * ZR, ZR)],
            sem.at[1 + i % (NB - 1)]))
    for cp in zcopies:
        cp.wait()
    plsc.subcore_barrier()

    def _chunk(j, carry):
        @pl.when(j >= NB)
        def _w():
            pltpu.make_async_copy(ones_v, deg_acc.at[dst_all.at[0]],
                                  sem.at[0]).wait()
        pltpu.async_copy(ones_v, deg_acc.at[dst_all.at[j]], sem.at[0],
                         add=True)
        return carry
    lax.fori_loop(0, NCHE, _chunk, 0)
    for _ in range(NB):
        pltpu.make_async_copy(ones_v, deg_acc.at[dst_all.at[0]],
                              sem.at[0]).wait()
    plsc.subcore_barrier()
    _copy_out(deg_acc, deg_hbm.at[cid], s)


_SC_MESH = plsc.VectorSubcoreMesh(core_axis_name="c", subcore_axis_name="s",
                                  num_cores=NC, num_subcores=NS)
_SC_PARAMS = pltpu.CompilerParams(use_tc_tiling_on_sc=False)


def _make_seg(split_cols, nch, w):
    return pl.kernel(
        functools.partial(_seg_body, split_cols, nch),
        out_type=[jax.ShapeDtypeStruct((NC, NN, w), jnp.float32)],
        mesh=_SC_MESH,
        scratch_types=[
            pltpu.VMEM((nch, CK), jnp.int32),
            pltpu.VMEM((nch, CK), jnp.int32),
        ] + [pltpu.VMEM((CK, w), jnp.float32) for _ in range(NB)] + [
            pltpu.VMEM((ZR, w), jnp.float32),
            pltpu.VMEM_SHARED((NPAD, w), jnp.float32),
            pltpu.SemaphoreType.DMA((NB,)),
            pltpu.SemaphoreType.DMA((NB,)),
        ],
        compiler_params=_SC_PARAMS)


_segw = _make_seg(True, NCHW, HW)
_sege = _make_seg(False, NCHE, 16)

_degk = pl.kernel(
    _deg_body,
    out_type=[jax.ShapeDtypeStruct((NC, NN, DEGW), jnp.float32)],
    mesh=_SC_MESH,
    scratch_types=[
        pltpu.VMEM((NCHE, CK), jnp.int32),
        pltpu.VMEM((CK, DEGW), jnp.float32),
        pltpu.VMEM((ZR, DEGW), jnp.float32),
        pltpu.VMEM_SHARED((NPAD, DEGW), jnp.float32),
        pltpu.SemaphoreType.DMA((NB,)),
    ],
    compiler_params=_SC_PARAMS)

BM = 1000


def _dot(a, b):
    return jnp.dot(a, b, preferred_element_type=jnp.float32)


def _mm_body(wn3_ref_last, h_ref, a_ref, d_ref, ws_ref, wn_ref, b_ref,
             out_ref, p3_ref=None):
    deg = jnp.maximum(d_ref[0, :, 0:1] + d_ref[1, :, 0:1], 1.0)
    hself = _dot(h_ref[0], ws_ref[:HW, :]) + _dot(h_ref[1], ws_ref[HW:, :])
    hn = (_dot(a_ref[0], wn_ref[:HW, :]) + _dot(a_ref[1], wn_ref[HW:, :])
          ) / deg
    y = jnp.maximum(hself + hn + b_ref[...], 0.0)
    out_ref[0] = y[:, :HW]
    out_ref[1] = y[:, HW:]
    if p3_ref is not None:
        p3_ref[...] = _dot(y, wn3_ref_last[...])


def _fin_body(h_ref, a_ref, d_ref, ws_ref, b_ref, out_ref):
    deg = jnp.maximum(d_ref[0, :, 0:1] + d_ref[1, :, 0:1], 1.0)
    hn = (a_ref[0] + a_ref[1]) / deg
    out_ref[...] = (_dot(h_ref[0], ws_ref[:HW, :])
                    + _dot(h_ref[1], ws_ref[HW:, :]) + hn + b_ref[...])


def _rows_spec(width):
    return pl.BlockSpec((BM, width), lambda i: (i, 0))


def _split_spec(width):
    return pl.BlockSpec((NC, BM, width), lambda i: (0, i, 0))


def _full_spec(r, width):
    return pl.BlockSpec((r, width), lambda i: (0, 0))


def _mm(h, a, d, ws, wn, b, wn3=None):
    body = functools.partial(_mm_body, None) if wn3 is None else _mm_body
    in_specs = [_split_spec(HW), _split_spec(HW), _split_spec(DEGW),
                _full_spec(128, 128), _full_spec(128, 128),
                _full_spec(1, 128)]
    out_specs = _split_spec(HW)
    out_shape = jax.ShapeDtypeStruct((NC, NN, HW), jnp.float32)
    args = [h, a, d, ws, wn, b]
    if wn3 is not None:
        in_specs = [_full_spec(128, 16)] + in_specs
        out_specs = [out_specs, _rows_spec(16)]
        out_shape = [out_shape, jax.ShapeDtypeStruct((NN, 16), jnp.float32)]
        args = [wn3] + args
    return pl.pallas_call(body, grid=(NN // BM,), in_specs=in_specs,
                          out_specs=out_specs, out_shape=out_shape)(*args)


def _fin(h, a, d, ws, b):
    return pl.pallas_call(
        _fin_body,
        grid=(NN // BM,),
        in_specs=[_split_spec(HW), _split_spec(16), _split_spec(DEGW),
                  _full_spec(128, 16), _full_spec(1, 16)],
        out_specs=_rows_spec(16),
        out_shape=jax.ShapeDtypeStruct((NN, 16), jnp.float32),
    )(h, a, d, ws, b)


def kernel(x, edge_index, W_self1, W_neigh1, b1, W_self2, W_neigh2, b2,
           W_self3, W_neigh3, b3):
    src = edge_index[0].astype(jnp.int32)
    dst = edge_index[1].astype(jnp.int32)
    pad_i = jnp.arange(EPAD - EE, dtype=jnp.int32)
    src_p = jnp.concatenate([src, pad_i % NN])
    dst_p = jnp.concatenate([dst, NN + (pad_i % (NPAD - NN))])
    src_w = src_p.reshape(NS, NCHW, CK)
    dst_w = dst_p.reshape(NS, NCHW, CK)
    src_e = src_p.reshape(NC * NS, NCHE, CK)
    dst_e = dst_p.reshape(NC * NS, NCHE, CK)
    xs = jnp.stack([x[:, :HW], x[:, HW:]])

    (deg,) = _degk(dst_e)
    (agg1,) = _segw(src_w, dst_w, xs)
    h1 = _mm(xs, agg1, deg, W_self1, W_neigh1, b1.reshape(1, 128))
    (agg2,) = _segw(src_w, dst_w, h1)
    h2, p3 = _mm(h1, agg2, deg, W_self2, W_neigh2, b2.reshape(1, 128),
                 wn3=W_neigh3)
    (agg3,) = _sege(src_e, dst_e, p3)
    out = _fin(h2, agg3, deg, W_self3, b3.reshape(1, 16))
    return out


# speedup vs baseline: 2.1894x; 1.0846x over previous
import functools

import jax
import jax.numpy as jnp
from jax import lax
from jax.experimental import pallas as pl
from jax.experimental.pallas import tpu as pltpu
from jax.experimental.pallas import tpu_sc as plsc

NN = 10000
EE = 320000
NC = 2
NS = 16
HW = 64
CK = 128
NB = 5
NPAD = 10240
RPT = NPAD // NS
TAIL = NN - (NS - 1) * RPT
ZR = 64
DEGW = 16
NCHW = 160
NCHE = 80
EPAD = NS * NCHW * CK
DUMP = NN + 16


def _zero_vmem(buf, rows, w):
    def _zrow(r, carry):
        for cc in range(w // 16):
            buf[r, pl.ds(cc * 16, 16)] = jnp.zeros((16,), jnp.float32)
        return carry
    lax.fori_loop(0, rows, _zrow, 0)


def _copy_out(acc, out_ref, s):
    @pl.when(s < NS - 1)
    def _full():
        pltpu.sync_copy(acc.at[pl.ds(s * RPT, RPT)],
                        out_ref.at[pl.ds(s * RPT, RPT)])

    @pl.when(s == NS - 1)
    def _tail():
        pltpu.sync_copy(acc.at[pl.ds((NS - 1) * RPT, TAIL)],
                        out_ref.at[pl.ds((NS - 1) * RPT, TAIL)])


def _seg_body(split_cols, nch, src_hbm, dst_hbm, h_hbm, out_hbm,
              src_all, dst_all, r0, r1, r2, r3, r4, zbuf, acc, sem_g, sem_s):
    rows = (r0, r1, r2, r3, r4)
    cid = lax.axis_index("c")
    s = lax.axis_index("s")
    tid = s if split_cols else cid * NS + s
    hsrc = h_hbm
    w = HW if split_cols else 16

    _zero_vmem(zbuf, ZR, w)
    zcopies = []
    for i in range(RPT // ZR):
        zcopies.append(pltpu.async_copy(
            zbuf, acc.at[pl.ds(s * RPT + i * ZR, ZR)], sem_g.at[i % NB]))
    zcopies.append(pltpu.async_copy(src_hbm.at[tid], src_all, sem_s.at[0]))
    zcopies.append(pltpu.async_copy(dst_hbm.at[tid], dst_all, sem_s.at[1]))
    for cp in zcopies:
        cp.wait()

    if split_cols:
        def _xform(r, carry):
            for cb in range(CK // 16):
                v = src_all[r, pl.ds(cb * 16, 16)]
                src_all[r, pl.ds(cb * 16, 16)] = v * 2 + cid
            return carry
        lax.fori_loop(0, nch, _xform, 0)
    plsc.subcore_barrier()

    def _issue_gather(c, b):
        pltpu.async_copy(hsrc.at[src_all.at[c]], rows[b], sem_g.at[b])

    def _wait_gather(b):
        pltpu.make_async_copy(hsrc.at[src_all.at[0]], rows[b],
                              sem_g.at[b]).wait()

    def _issue_scatter(c, b):
        pltpu.async_copy(rows[b], acc.at[dst_all.at[c]], sem_s.at[b],
                         add=True)

    def _wait_scatter(b):
        pltpu.make_async_copy(rows[b], acc.at[dst_all.at[0]],
                              sem_s.at[b]).wait()

    def _group(j, carry):
        for b in range(NB):
            c = j * NB + b

            @pl.when(j >= 1)
            def _ws():
                _wait_scatter(b)
            _issue_gather(c, b)

            pb = (b - 1) % NB
            if b == 0:
                @pl.when(j >= 1)
                def _sc():
                    _wait_gather(pb)
                    _issue_scatter(j * NB - 1, pb)
            else:
                _wait_gather(pb)
                _issue_scatter(c - 1, pb)
        return carry
    lax.fori_loop(0, nch // NB, _group, 0)

    _wait_gather(NB - 1)
    _issue_scatter(nch - 1, NB - 1)
    for b in range(NB):
        _wait_scatter(b)
    plsc.subcore_barrier()

    _copy_out(acc, out_hbm.at[cid], s)


def _deg_body(dst_hbm, deg_hbm, dst_all, ones_v, zbuf, deg_acc, sem):
    cid = lax.axis_index("c")
    s = lax.axis_index("s")
    tid = cid * NS + s

    _zero_vmem(zbuf, ZR, DEGW)

    def _orow(r, carry):
        ones_v[r, pl.ds(0, 16)] = jnp.ones((16,), jnp.float32)
        return carry
    lax.fori_loop(0, CK, _orow, 0)

    zcopies = [pltpu.async_copy(dst_hbm.at[tid], dst_all, sem.at[0])]
    for i in range(RPT // ZR):
        zcopies.append(pltpu.async_copy(
            zbuf, deg_acc.at[pl.ds(s * RPT + i * ZR, ZR)],
            sem.at[1 + i % (NB - 1)]))
    for cp in zcopies:
        cp.wait()
    plsc.subcore_barrier()

    def _chunk(j, carry):
        @pl.when(j >= NB)
        def _w():
            pltpu.make_async_copy(ones_v, deg_acc.at[dst_all.at[0]],
                                  sem.at[0]).wait()
        pltpu.async_copy(ones_v, deg_acc.at[dst_all.at[j]], sem.at[0],
                         add=True)
        return carry
    lax.fori_loop(0, NCHE, _chunk, 0)
    for _ in range(NB):
        pltpu.make_async_copy(ones_v, deg_acc.at[dst_all.at[0]],
                              sem.at[0]).wait()
    plsc.subcore_barrier()
    _copy_out(deg_acc, deg_hbm.at[cid], s)


_SC_MESH = plsc.VectorSubcoreMesh(core_axis_name="c", subcore_axis_name="s",
                                  num_cores=NC, num_subcores=NS)
_SC_PARAMS = pltpu.CompilerParams(use_tc_tiling_on_sc=False)


def _make_seg(split_cols, nch, w):
    return pl.kernel(
        functools.partial(_seg_body, split_cols, nch),
        out_type=[jax.ShapeDtypeStruct((NC, NN, w), jnp.float32)],
        name="segw" if split_cols else "sege",
        mesh=_SC_MESH,
        scratch_types=[
            pltpu.VMEM((nch, CK), jnp.int32),
            pltpu.VMEM((nch, CK), jnp.int32),
        ] + [pltpu.VMEM((CK, w), jnp.float32) for _ in range(NB)] + [
            pltpu.VMEM((ZR, w), jnp.float32),
            pltpu.VMEM_SHARED((NPAD, w), jnp.float32),
            pltpu.SemaphoreType.DMA((NB,)),
            pltpu.SemaphoreType.DMA((NB,)),
        ],
        compiler_params=_SC_PARAMS)


_segw = _make_seg(True, NCHW, HW)
_sege = _make_seg(False, NCHE, 16)

_degk = pl.kernel(
    _deg_body,
    out_type=[jax.ShapeDtypeStruct((NC, NN, DEGW), jnp.float32)],
    mesh=_SC_MESH,
    scratch_types=[
        pltpu.VMEM((NCHE, CK), jnp.int32),
        pltpu.VMEM((CK, DEGW), jnp.float32),
        pltpu.VMEM((ZR, DEGW), jnp.float32),
        pltpu.VMEM_SHARED((NPAD, DEGW), jnp.float32),
        pltpu.SemaphoreType.DMA((NB,)),
    ],
    compiler_params=_SC_PARAMS)

BM = 1000


def _dot(a, b):
    return jnp.dot(a, b, preferred_element_type=jnp.float32)


def _mm_body(wn3_ref_last, h_ref, a_ref, d_ref, ws_ref, wn_ref, b_ref,
             out_ref, p3_ref=None):
    deg = jnp.maximum(d_ref[0, :, 0:1] + d_ref[1, :, 0:1], 1.0)
    hn = (_dot(a_ref[0], wn_ref[:HW, :]) + _dot(a_ref[1], wn_ref[HW:, :])
          ) / deg
    y = jnp.maximum(_dot(h_ref[...], ws_ref[...]) + hn + b_ref[...], 0.0)
    out_ref[...] = y
    if p3_ref is not None:
        p3_ref[...] = _dot(y, wn3_ref_last[...])


def _fin_body(h_ref, a_ref, d_ref, ws_ref, b_ref, out_ref):
    deg = jnp.maximum(d_ref[0, :, 0:1] + d_ref[1, :, 0:1], 1.0)
    hn = (a_ref[0] + a_ref[1]) / deg
    out_ref[...] = _dot(h_ref[...], ws_ref[...]) + hn + b_ref[...]


def _rows_spec(width):
    return pl.BlockSpec((BM, width), lambda i: (i, 0))


def _split_spec(width):
    return pl.BlockSpec((NC, BM, width), lambda i: (0, i, 0))


def _full_spec(r, width):
    return pl.BlockSpec((r, width), lambda i: (0, 0))


def _mm(h, a, d, ws, wn, b, wn3=None):
    body = functools.partial(_mm_body, None) if wn3 is None else _mm_body
    in_specs = [_rows_spec(128), _split_spec(HW), _split_spec(DEGW),
                _full_spec(128, 128), _full_spec(128, 128),
                _full_spec(1, 128)]
    out_specs = _rows_spec(128)
    out_shape = jax.ShapeDtypeStruct((NN, 128), jnp.float32)
    args = [h, a, d, ws, wn, b]
    if wn3 is not None:
        in_specs = [_full_spec(128, 16)] + in_specs
        out_specs = [out_specs, _rows_spec(16)]
        out_shape = [out_shape, jax.ShapeDtypeStruct((NN, 16), jnp.float32)]
        args = [wn3] + args
    return pl.pallas_call(body, grid=(NN // BM,), in_specs=in_specs,
                          out_specs=out_specs, out_shape=out_shape)(*args)


def _fin(h, a, d, ws, b):
    return pl.pallas_call(
        _fin_body,
        grid=(NN // BM,),
        in_specs=[_rows_spec(128), _split_spec(16), _split_spec(DEGW),
                  _full_spec(128, 16), _full_spec(1, 16)],
        out_specs=_rows_spec(16),
        out_shape=jax.ShapeDtypeStruct((NN, 16), jnp.float32),
    )(h, a, d, ws, b)


def kernel(x, edge_index, W_self1, W_neigh1, b1, W_self2, W_neigh2, b2,
           W_self3, W_neigh3, b3):
    src = edge_index[0].astype(jnp.int32)
    dst = edge_index[1].astype(jnp.int32)
    pad_i = jnp.arange(EPAD - EE, dtype=jnp.int32)
    src_p = jnp.concatenate([src, pad_i % NN])
    dst_p = jnp.concatenate([dst, NN + (pad_i % (NPAD - NN))])
    src_w = src_p.reshape(NS, NCHW, CK)
    dst_w = dst_p.reshape(NS, NCHW, CK)
    src_e = src_p.reshape(NC * NS, NCHE, CK)
    dst_e = dst_p.reshape(NC * NS, NCHE, CK)

    (deg,) = _degk(dst_e)
    (agg1,) = _segw(src_w, dst_w, x.reshape(2 * NN, HW))
    h1 = _mm(x, agg1, deg, W_self1, W_neigh1, b1.reshape(1, 128))
    (agg2,) = _segw(src_w, dst_w, h1.reshape(2 * NN, HW))
    h2, p3 = _mm(h1, agg2, deg, W_self2, W_neigh2, b2.reshape(1, 128),
                 wn3=W_neigh3)
    (agg3,) = _sege(src_e, dst_e, p3)
    out = _fin(h2, agg3, deg, W_self3, b3.reshape(1, 16))
    return out
